# baseline (speedup 1.0000x reference)
import functools

import jax
import jax.numpy as jnp
from jax import lax
from jax.experimental import pallas as pl
from jax.experimental.pallas import tpu as pltpu
from jax.experimental.pallas import tpu_sc as plsc

VOCAB = 100000
DIM = 128
BATCH = 4096
HIST = 50
HIST_PAD = 56

_info = plsc.get_sparse_core_info()
_NC, _NS = _info.num_cores, _info.num_subcores
NW = _NC * _NS
TOTAL = BATCH * HIST_PAD
PER_W = TOTAL // NW
CHUNK = 128
NCHUNK = PER_W // CHUNK
NBUF = 4
NGROUP = NCHUNK // NBUF


def _emb_body(table, idx, out, idx_v, *rest):
    bufs = rest[:NBUF]
    gsems = rest[NBUF:2 * NBUF]
    osems = rest[2 * NBUF:3 * NBUF]
    wid = lax.axis_index("s") * _NC + lax.axis_index("c")
    pltpu.sync_copy(idx.at[wid], idx_v)

    for b in range(NBUF):
        pltpu.async_copy(table.at[idx_v.at[b]], bufs[b], gsems[b])

    def group(g, carry):
        for b in range(NBUF):
            c = g * NBUF + b
            pltpu.make_async_copy(table.at[idx_v.at[0]], bufs[b], gsems[b]).wait()
            pltpu.async_copy(bufs[b], out.at[wid, c], osems[b])

        @pl.when(g < NGROUP - 1)
        def _():
            for b in range(NBUF):
                pltpu.make_async_copy(bufs[b], out.at[wid, 0], osems[b]).wait()
                pltpu.async_copy(table.at[idx_v.at[(g + 1) * NBUF + b]],
                                 bufs[b], gsems[b])
        return carry

    lax.fori_loop(0, NGROUP, group, 0)
    for b in range(NBUF):
        pltpu.make_async_copy(bufs[b], out.at[wid, 0], osems[b]).wait()


_emb_call = functools.partial(
    pl.kernel,
    out_type=jax.ShapeDtypeStruct((NW, NCHUNK, CHUNK, DIM), jnp.float32),
    mesh=plsc.VectorSubcoreMesh(core_axis_name="c", subcore_axis_name="s"),
    scratch_types=(
        [pltpu.VMEM((NCHUNK, CHUNK), jnp.int32)]
        + [pltpu.VMEM((CHUNK, DIM), jnp.float32) for _ in range(NBUF)]
        + [pltpu.SemaphoreType.DMA for _ in range(2 * NBUF)]
    ),
)(_emb_body)


def kernel(inputs, embeddings):
    idx = inputs.astype(jnp.int32)
    idx = jnp.pad(idx, ((0, 0), (0, HIST_PAD - HIST)))
    idx = idx.reshape(NW, NCHUNK, CHUNK)
    out = _emb_call(embeddings, idx)
    return out.reshape(BATCH, HIST_PAD, DIM)[:, :HIST, :]


# speedup vs baseline: 6.4090x; 6.4090x over previous
import functools

import jax
import jax.numpy as jnp
from jax import lax
from jax.experimental import pallas as pl
from jax.experimental.pallas import tpu as pltpu
from jax.experimental.pallas import tpu_sc as plsc

VOCAB = 100000
DIM = 128
BATCH = 4096
HIST = 50
HIST_PAD = 56

_info = plsc.get_sparse_core_info()
_NC, _NS = _info.num_cores, _info.num_subcores
NW = _NC * _NS
TOTAL = BATCH * HIST_PAD
PER_W = TOTAL // NW
CHUNK = 128
NCHUNK = PER_W // CHUNK
NBUF = 4
NGROUP = NCHUNK // NBUF


def _emb_body(table, idx, out, idx_v, *rest):
    bufs = rest[:NBUF]
    gsems = rest[NBUF:2 * NBUF]
    osems = rest[2 * NBUF:3 * NBUF]
    wid = lax.axis_index("s") * _NC + lax.axis_index("c")
    pltpu.sync_copy(idx.at[wid], idx_v)

    for b in range(NBUF):
        pltpu.async_copy(table.at[idx_v.at[b]], bufs[b], gsems[b])

    def group(g, carry):
        for b in range(NBUF):
            c = g * NBUF + b
            pltpu.make_async_copy(table.at[idx_v.at[0]], bufs[b], gsems[b]).wait()
            pltpu.async_copy(bufs[b], out.at[wid, c], osems[b])

        @pl.when(g < NGROUP - 1)
        def _():
            for b in range(NBUF):
                pltpu.make_async_copy(bufs[b], out.at[wid, 0], osems[b]).wait()
                pltpu.async_copy(table.at[idx_v.at[(g + 1) * NBUF + b]],
                                 bufs[b], gsems[b])
        return carry

    lax.fori_loop(0, NGROUP, group, 0)
    for b in range(NBUF):
        pltpu.make_async_copy(bufs[b], out.at[wid, 0], osems[b]).wait()


_emb_call = functools.partial(
    pl.kernel,
    out_type=jax.ShapeDtypeStruct((NW, NCHUNK, CHUNK, DIM), jnp.float32),
    mesh=plsc.VectorSubcoreMesh(core_axis_name="c", subcore_axis_name="s"),
    scratch_types=(
        [pltpu.VMEM((NCHUNK, CHUNK), jnp.int32)]
        + [pltpu.VMEM((CHUNK, DIM), jnp.float32) for _ in range(NBUF)]
        + [pltpu.SemaphoreType.DMA for _ in range(2 * NBUF)]
    ),
)(_emb_body)


def kernel(inputs, embeddings):
    idx = inputs.astype(jnp.int32)
    idx = jnp.concatenate([idx, idx[:, : HIST_PAD - HIST]], axis=1)
    idx = idx.reshape(NW, NCHUNK, CHUNK)
    out = _emb_call(embeddings, idx)
    return out.reshape(BATCH, HIST_PAD, DIM)[:, :HIST, :]


# speedup vs baseline: 7.7394x; 1.2076x over previous
import functools

import jax
import jax.numpy as jnp
from jax import lax
from jax.experimental import pallas as pl
from jax.experimental.pallas import tpu as pltpu
from jax.experimental.pallas import tpu_sc as plsc

VOCAB = 100000
DIM = 128
BATCH = 4096
HIST = 50

_info = plsc.get_sparse_core_info()
_NC, _NS = _info.num_cores, _info.num_subcores
NW = _NC * _NS
BPW = BATCH // NW
NBUF = 8
NGROUP = BPW // NBUF


def _emb_body(table, idx, out, raw_v, *rest):
    bufs = rest[:NBUF]
    gsems = rest[NBUF:2 * NBUF]
    osems = rest[2 * NBUF:3 * NBUF]
    wid = lax.axis_index("s") * _NC + lax.axis_index("c")
    base = wid * BPW
    pltpu.sync_copy(idx.at[pl.ds(base, BPW)], raw_v)

    def gather(bb, b):
        pltpu.async_copy(table.at[raw_v.at[bb, pl.ds(0, HIST)]],
                         bufs[b], gsems[b])

    for b in range(NBUF):
        gather(b, b)

    def group(g, carry):
        for b in range(NBUF):
            bb = g * NBUF + b
            pltpu.make_async_copy(table.at[raw_v.at[0, pl.ds(0, HIST)]],
                                  bufs[b], gsems[b]).wait()
            pltpu.async_copy(bufs[b], out.at[base + bb], osems[b])

        @pl.when(g < NGROUP - 1)
        def _():
            for b in range(NBUF):
                pltpu.make_async_copy(bufs[b], out.at[0], osems[b]).wait()
                gather((g + 1) * NBUF + b, b)
        return carry

    lax.fori_loop(0, NGROUP, group, 0)
    for b in range(NBUF):
        pltpu.make_async_copy(bufs[b], out.at[0], osems[b]).wait()


_emb_call = functools.partial(
    pl.kernel,
    out_type=jax.ShapeDtypeStruct((BATCH, HIST, DIM), jnp.float32),
    mesh=plsc.VectorSubcoreMesh(core_axis_name="c", subcore_axis_name="s"),
    compiler_params=pltpu.CompilerParams(use_tc_tiling_on_sc=True),
    scratch_types=(
        [pltpu.VMEM((BPW, 128), jnp.int32)]
        + [pltpu.VMEM((HIST, DIM), jnp.float32) for _ in range(NBUF)]
        + [pltpu.SemaphoreType.DMA for _ in range(2 * NBUF)]
    ),
)(_emb_body)


def kernel(inputs, embeddings):
    idx = jnp.pad(inputs.astype(jnp.int32), ((0, 0), (0, 128 - HIST)),
                  mode="wrap")
    return _emb_call(embeddings, idx)
